# baseline (device time: 168167 ns/iter reference)
import jax
import jax.numpy as jnp
from jax import lax
from jax.experimental import pallas as pl
from jax.experimental.pallas import tpu as pltpu

N_DEV = 16
SQ = 2048
D_MODEL = 1024
H_PER = 8
DH = 128
H_SLICE = H_PER * DH
CHUNK = SQ // N_DEV
BLK = 64
SCALE = 0.08838834764831843

N_STEP = N_DEV - 1


U = 64


def _allreduce_body(xt_ref, xb_ref, out_ref, g_ref,
                    a0, a1, a2, a3, b0, b1, b2, b3,
                    rs_send_sems, rs_recv_sems, ag_send_sems, ag_recv_sems):
    pos = lax.axis_index("i")
    w = lax.rem(pos, 4)
    z = pos // 4
    bits = {
        "x": jnp.where((w == 1) | (w == 2), 1, 0).astype(jnp.int32),
        "y": w // 2,
        "z0": lax.rem(z, 2),
        "z1": z // 2,
    }
    partners = {
        "x": pos + 1 - 2 * lax.rem(w, 2),
        "y": pos + 3 - 2 * w,
        "z0": pos + (1 - 2 * bits["z0"]) * 4,
        "z1": pos + (1 - 2 * bits["z1"]) * 8,
    }
    order_a = ["x", "y", "z0", "z1"]
    order_b = ["y", "x", "z1", "z0"]
    bufs_a = [a0, a1, a2, a3]
    bufs_b = [b0, b1, b2, b3]

    out_ref[0:SQ // 2, :] = xt_ref[...]
    out_ref[SQ // 2:SQ, :] = xb_ref[...]

    act = {"a": jnp.int32(0), "b": jnp.int32(0)}
    base = {"a": 0, "b": 16}
    for k in range(4):
        h = 8 >> k
        rdmas = []
        for half, order, bufs, soff in (("a", order_a, bufs_a, 0),
                                        ("b", order_b, bufs_b, 4)):
            d = order[k]
            b, partner = bits[d], partners[d]
            keep = act[half] + b * h
            send = act[half] + (1 - b) * h
            s_sl = pl.ds((base[half] + send) * U, h * U)
            g_ref[s_sl, :] = out_ref[s_sl, :].astype(jnp.bfloat16)
            rdma = pltpu.make_async_remote_copy(
                src_ref=g_ref.at[s_sl, :],
                dst_ref=bufs[k],
                send_sem=rs_send_sems.at[k + soff],
                recv_sem=rs_recv_sems.at[k + soff],
                device_id=(partner,),
                device_id_type=pl.DeviceIdType.MESH,
            )
            rdma.start()
            rdmas.append((rdma, half, keep, bufs[k]))
        for rdma, half, keep, rbuf in rdmas:
            rdma.wait()
            k_sl = pl.ds((base[half] + keep) * U, rbuf.shape[0])
            out_ref[k_sl, :] = out_ref[k_sl, :] + rbuf[...].astype(jnp.float32)
            act[half] = keep

    for half in ("a", "b"):
        o_sl = pl.ds((base[half] + act[half]) * U, U)
        g_ref[o_sl, :] = out_ref[o_sl, :].astype(jnp.bfloat16)

    for k in range(4):
        sz = 1 << k
        rdmas = []
        for half, order, soff in (("a", order_a, 0), ("b", order_b, 4)):
            d = order[3 - k]
            b, partner = bits[d], partners[d]
            new = act[half] - b * sz
            other = new + (1 - b) * sz
            m_sl = pl.ds((base[half] + act[half]) * U, sz * U)
            rdma = pltpu.make_async_remote_copy(
                src_ref=g_ref.at[m_sl, :],
                dst_ref=g_ref.at[m_sl, :],
                send_sem=ag_send_sems.at[k + soff],
                recv_sem=ag_recv_sems.at[k + soff],
                device_id=(partner,),
                device_id_type=pl.DeviceIdType.MESH,
            )
            rdma.start()
            rdmas.append((rdma, half, new, other))
        for rdma, half, new, other in rdmas:
            rdma.wait()
            o_sl = pl.ds((base[half] + other) * U, sz * U)
            out_ref[o_sl, :] = g_ref[o_sl, :].astype(jnp.float32)
            act[half] = new


def _allreduce_body_single(x_ref, out_ref, g_ref, rb0, rb1, rb2, rb3,
                    rs_send_sems, rs_recv_sems, ag_send_sems, ag_recv_sems):
    pos = lax.axis_index("i")
    w = lax.rem(pos, 4)
    z = pos // 4
    b_x = jnp.where((w == 1) | (w == 2), 1, 0).astype(jnp.int32)
    b_y = w // 2
    b_z0 = lax.rem(z, 2)
    b_z1 = z // 2
    p_x = pos + 1 - 2 * lax.rem(w, 2)
    p_y = pos + 3 - 2 * w
    p_z0 = pos + (1 - 2 * b_z0) * 4
    p_z1 = pos + (1 - 2 * b_z1) * 8

    out_ref[...] = x_ref[...]

    active = jnp.int32(0)
    rs_steps = [(8, b_x, p_x, rb0), (4, b_y, p_y, rb1),
                (2, b_z0, p_z0, rb2), (1, b_z1, p_z1, rb3)]
    for k, (h, b, partner, rbuf) in enumerate(rs_steps):
        keep = active + b * h
        send = active + (1 - b) * h
        s_sl = pl.ds(send * CHUNK, h * CHUNK)
        g_ref[s_sl, :] = out_ref[s_sl, :].astype(jnp.bfloat16)
        rdma = pltpu.make_async_remote_copy(
            src_ref=g_ref.at[s_sl, :],
            dst_ref=rbuf,
            send_sem=rs_send_sems.at[k],
            recv_sem=rs_recv_sems.at[k],
            device_id=(partner,),
            device_id_type=pl.DeviceIdType.MESH,
        )
        rdma.start()
        rdma.wait()
        k_sl = pl.ds(keep * CHUNK, h * CHUNK)
        out_ref[k_sl, :] = out_ref[k_sl, :] + rbuf[...].astype(jnp.float32)
        active = keep

    own_sl = pl.ds(active * CHUNK, CHUNK)
    g_ref[own_sl, :] = out_ref[own_sl, :].astype(jnp.bfloat16)

    ag_steps = [(1, b_z1, p_z1), (2, b_z0, p_z0),
                (4, b_y, p_y), (8, b_x, p_x)]
    for k, (sz, b, partner) in enumerate(ag_steps):
        new = active - b * sz
        other = new + (1 - b) * sz
        my_sl = pl.ds(active * CHUNK, sz * CHUNK)
        rdma = pltpu.make_async_remote_copy(
            src_ref=g_ref.at[my_sl, :],
            dst_ref=g_ref.at[my_sl, :],
            send_sem=ag_send_sems.at[k],
            recv_sem=ag_recv_sems.at[k],
            device_id=(partner,),
            device_id_type=pl.DeviceIdType.MESH,
        )
        rdma.start()
        rdma.wait()
        o_sl = pl.ds(other * CHUNK, sz * CHUNK)
        out_ref[o_sl, :] = g_ref[o_sl, :].astype(jnp.float32)
        active = new


def _ring_allreduce(partial_top, partial_bot):
    return pl.pallas_call(
        _allreduce_body,
        out_shape=jax.ShapeDtypeStruct((SQ, D_MODEL), jnp.float32),
        in_specs=[pl.BlockSpec(memory_space=pltpu.VMEM),
                  pl.BlockSpec(memory_space=pltpu.VMEM)],
        out_specs=pl.BlockSpec(memory_space=pltpu.VMEM),
        scratch_shapes=[
            pltpu.VMEM((SQ, D_MODEL), jnp.bfloat16),
            pltpu.VMEM((8 * U, D_MODEL), jnp.bfloat16),
            pltpu.VMEM((4 * U, D_MODEL), jnp.bfloat16),
            pltpu.VMEM((2 * U, D_MODEL), jnp.bfloat16),
            pltpu.VMEM((1 * U, D_MODEL), jnp.bfloat16),
            pltpu.VMEM((8 * U, D_MODEL), jnp.bfloat16),
            pltpu.VMEM((4 * U, D_MODEL), jnp.bfloat16),
            pltpu.VMEM((2 * U, D_MODEL), jnp.bfloat16),
            pltpu.VMEM((1 * U, D_MODEL), jnp.bfloat16),
            pltpu.SemaphoreType.DMA((8,)),
            pltpu.SemaphoreType.DMA((8,)),
            pltpu.SemaphoreType.DMA((8,)),
            pltpu.SemaphoreType.DMA((8,)),
        ],
    )(partial_top, partial_bot)


QT = 256
NQ = SQ // QT


def _make_attn_body(base_tile, kv_len):
    def _attn_body(x_ref, wq_ref, k_ref, v_ref, o_ref):
        i = pl.program_id(1) + base_tile
        q = (jnp.dot(x_ref[...], wq_ref[...],
                     preferred_element_type=jnp.float32)
             * SCALE).astype(jnp.bfloat16)
        s = lax.dot_general(q, k_ref[...], (((1,), (1,)), ((), ())),
                            preferred_element_type=jnp.float32)
        r = lax.broadcasted_iota(jnp.int32, (QT, 1), 0) + i * QT
        thr = r - lax.rem(r, BLK) + BLK
        c = lax.broadcasted_iota(jnp.int32, (QT, kv_len), 1)
        p = jnp.exp(jnp.where(c < thr, s, -1e4))
        l = jnp.sum(p, axis=1, keepdims=True)
        o = lax.dot_general(p.astype(jnp.bfloat16), v_ref[...],
                            (((1,), (0,)), ((), ())),
                            preferred_element_type=jnp.float32)
        o_ref[...] = (o / l).astype(o_ref.dtype)
    return _attn_body


def _attention_ranged(x_part, wq_l, k, v, base_tile, kv_len):
    ntiles = x_part.shape[0] // QT
    return pl.pallas_call(
        _make_attn_body(base_tile, kv_len),
        out_shape=jax.ShapeDtypeStruct((ntiles * QT, H_SLICE), jnp.bfloat16),
        grid=(H_PER, ntiles),
        in_specs=[
            pl.BlockSpec((QT, D_MODEL), lambda h, i: (i, 0)),
            pl.BlockSpec((D_MODEL, DH), lambda h, i: (0, h)),
            pl.BlockSpec((kv_len, DH), lambda h, i: (0, h)),
            pl.BlockSpec((kv_len, DH), lambda h, i: (0, h)),
        ],
        out_specs=pl.BlockSpec((QT, DH), lambda h, i: (i, h)),
    )(x_part, wq_l, k, v)


def kernel(x, Wq, K_ext, V_ext, Wo):
    pos = lax.axis_index("i")
    bf = jnp.bfloat16

    x2 = x[0].astype(bf)
    Wq_l = lax.dynamic_slice_in_dim(Wq, pos * H_SLICE, H_SLICE,
                                    axis=1).astype(bf)
    Wo_l = lax.dynamic_slice_in_dim(Wo, pos * H_SLICE, H_SLICE,
                                    axis=0).astype(bf)
    k = K_ext[0].reshape(SQ, H_SLICE).astype(bf)
    v = V_ext[0].reshape(SQ, H_SLICE).astype(bf)

    ctx_top = _attention_ranged(x2[:SQ // 2], Wq_l, k, v, 0, SQ // 2)
    ctx_bot = _attention_ranged(x2[SQ // 2:], Wq_l, k, v, NQ // 2, SQ)
    partial_top = jnp.dot(ctx_top, Wo_l,
                          preferred_element_type=jnp.float32)
    partial_bot = jnp.dot(ctx_bot, Wo_l,
                          preferred_element_type=jnp.float32)

    out = _ring_allreduce(partial_top, partial_bot)
    return out[None, :, :]


# device time: 161024 ns/iter; 1.0444x vs baseline; 1.0444x over previous
import jax
import jax.numpy as jnp
from jax import lax
from jax.experimental import pallas as pl
from jax.experimental.pallas import tpu as pltpu

N_DEV = 16
SQ = 2048
D_MODEL = 1024
H_PER = 8
DH = 128
H_SLICE = H_PER * DH
CHUNK = SQ // N_DEV
BLK = 64
SCALE = 0.08838834764831843

N_STEP = N_DEV - 1


U = 64


def _allreduce_body(xt_ref, xb_ref, out_ref, g_ref,
                    a0, a1, a2, a3, b0, b1, b2, b3,
                    rs_send_sems, rs_recv_sems, ag_send_sems, ag_recv_sems):
    pos = lax.axis_index("i")
    w = lax.rem(pos, 4)
    z = pos // 4
    bits = {
        "x": jnp.where((w == 1) | (w == 2), 1, 0).astype(jnp.int32),
        "y": w // 2,
        "z0": lax.rem(z, 2),
        "z1": z // 2,
    }
    partners = {
        "x": pos + 1 - 2 * lax.rem(w, 2),
        "y": pos + 3 - 2 * w,
        "z0": pos + (1 - 2 * bits["z0"]) * 4,
        "z1": pos + (1 - 2 * bits["z1"]) * 8,
    }
    order_a = ["x", "y", "z0", "z1"]
    order_b = ["y", "x", "z1", "z0"]
    bufs_a = [a0, a1, a2, a3]
    bufs_b = [b0, b1, b2, b3]

    out_ref[0:SQ // 2, :] = xt_ref[...]
    out_ref[SQ // 2:SQ, :] = xb_ref[...]

    act = {"a": jnp.int32(0), "b": jnp.int32(0)}
    base = {"a": 0, "b": 16}
    for k in range(4):
        h = 8 >> k
        rdmas = []
        for half, order, bufs, soff in (("a", order_a, bufs_a, 0),
                                        ("b", order_b, bufs_b, 4)):
            d = order[k]
            b, partner = bits[d], partners[d]
            keep = act[half] + b * h
            send = act[half] + (1 - b) * h
            s_sl = pl.ds((base[half] + send) * U, h * U)
            g_ref[s_sl, :] = out_ref[s_sl, :].astype(jnp.bfloat16)
            rdma = pltpu.make_async_remote_copy(
                src_ref=g_ref.at[s_sl, :],
                dst_ref=bufs[k],
                send_sem=rs_send_sems.at[k + soff],
                recv_sem=rs_recv_sems.at[k + soff],
                device_id=(partner,),
                device_id_type=pl.DeviceIdType.MESH,
            )
            rdma.start()
            rdmas.append((rdma, half, keep, bufs[k]))
        for rdma, half, keep, rbuf in rdmas:
            rdma.wait()
            k_sl = pl.ds((base[half] + keep) * U, rbuf.shape[0])
            out_ref[k_sl, :] = out_ref[k_sl, :] + rbuf[...].astype(jnp.float32)
            act[half] = keep

    for half in ("a", "b"):
        o_sl = pl.ds((base[half] + act[half]) * U, U)
        g_ref[o_sl, :] = out_ref[o_sl, :].astype(jnp.bfloat16)

    for k in range(4):
        sz = 1 << k
        rdmas = []
        for half, order, soff in (("a", order_a, 0), ("b", order_b, 4)):
            d = order[3 - k]
            b, partner = bits[d], partners[d]
            new = act[half] - b * sz
            other = new + (1 - b) * sz
            m_sl = pl.ds((base[half] + act[half]) * U, sz * U)
            rdma = pltpu.make_async_remote_copy(
                src_ref=g_ref.at[m_sl, :],
                dst_ref=g_ref.at[m_sl, :],
                send_sem=ag_send_sems.at[k + soff],
                recv_sem=ag_recv_sems.at[k + soff],
                device_id=(partner,),
                device_id_type=pl.DeviceIdType.MESH,
            )
            rdma.start()
            rdmas.append((rdma, half, new, other))
        for rdma, half, new, other in rdmas:
            rdma.wait()
            o_sl = pl.ds((base[half] + other) * U, sz * U)
            out_ref[o_sl, :] = g_ref[o_sl, :].astype(jnp.float32)
            act[half] = new


def _allreduce_body_single(x_ref, out_ref, g_ref, rb0, rb1, rb2, rb3,
                    rs_send_sems, rs_recv_sems, ag_send_sems, ag_recv_sems):
    pos = lax.axis_index("i")
    w = lax.rem(pos, 4)
    z = pos // 4
    b_x = jnp.where((w == 1) | (w == 2), 1, 0).astype(jnp.int32)
    b_y = w // 2
    b_z0 = lax.rem(z, 2)
    b_z1 = z // 2
    p_x = pos + 1 - 2 * lax.rem(w, 2)
    p_y = pos + 3 - 2 * w
    p_z0 = pos + (1 - 2 * b_z0) * 4
    p_z1 = pos + (1 - 2 * b_z1) * 8

    out_ref[...] = x_ref[...]

    active = jnp.int32(0)
    rs_steps = [(8, b_x, p_x, rb0), (4, b_y, p_y, rb1),
                (2, b_z0, p_z0, rb2), (1, b_z1, p_z1, rb3)]
    for k, (h, b, partner, rbuf) in enumerate(rs_steps):
        keep = active + b * h
        send = active + (1 - b) * h
        s_sl = pl.ds(send * CHUNK, h * CHUNK)
        g_ref[s_sl, :] = out_ref[s_sl, :].astype(jnp.bfloat16)
        rdma = pltpu.make_async_remote_copy(
            src_ref=g_ref.at[s_sl, :],
            dst_ref=rbuf,
            send_sem=rs_send_sems.at[k],
            recv_sem=rs_recv_sems.at[k],
            device_id=(partner,),
            device_id_type=pl.DeviceIdType.MESH,
        )
        rdma.start()
        rdma.wait()
        k_sl = pl.ds(keep * CHUNK, h * CHUNK)
        out_ref[k_sl, :] = out_ref[k_sl, :] + rbuf[...].astype(jnp.float32)
        active = keep

    own_sl = pl.ds(active * CHUNK, CHUNK)
    g_ref[own_sl, :] = out_ref[own_sl, :].astype(jnp.bfloat16)

    ag_steps = [(1, b_z1, p_z1), (2, b_z0, p_z0),
                (4, b_y, p_y), (8, b_x, p_x)]
    for k, (sz, b, partner) in enumerate(ag_steps):
        new = active - b * sz
        other = new + (1 - b) * sz
        my_sl = pl.ds(active * CHUNK, sz * CHUNK)
        rdma = pltpu.make_async_remote_copy(
            src_ref=g_ref.at[my_sl, :],
            dst_ref=g_ref.at[my_sl, :],
            send_sem=ag_send_sems.at[k],
            recv_sem=ag_recv_sems.at[k],
            device_id=(partner,),
            device_id_type=pl.DeviceIdType.MESH,
        )
        rdma.start()
        rdma.wait()
        o_sl = pl.ds(other * CHUNK, sz * CHUNK)
        out_ref[o_sl, :] = g_ref[o_sl, :].astype(jnp.float32)
        active = new


def _ring_allreduce(partial_top, partial_bot):
    return pl.pallas_call(
        _allreduce_body,
        out_shape=jax.ShapeDtypeStruct((SQ, D_MODEL), jnp.float32),
        in_specs=[pl.BlockSpec(memory_space=pltpu.VMEM),
                  pl.BlockSpec(memory_space=pltpu.VMEM)],
        out_specs=pl.BlockSpec(memory_space=pltpu.VMEM),
        scratch_shapes=[
            pltpu.VMEM((SQ, D_MODEL), jnp.bfloat16),
            pltpu.VMEM((8 * U, D_MODEL), jnp.bfloat16),
            pltpu.VMEM((4 * U, D_MODEL), jnp.bfloat16),
            pltpu.VMEM((2 * U, D_MODEL), jnp.bfloat16),
            pltpu.VMEM((1 * U, D_MODEL), jnp.bfloat16),
            pltpu.VMEM((8 * U, D_MODEL), jnp.bfloat16),
            pltpu.VMEM((4 * U, D_MODEL), jnp.bfloat16),
            pltpu.VMEM((2 * U, D_MODEL), jnp.bfloat16),
            pltpu.VMEM((1 * U, D_MODEL), jnp.bfloat16),
            pltpu.SemaphoreType.DMA((8,)),
            pltpu.SemaphoreType.DMA((8,)),
            pltpu.SemaphoreType.DMA((8,)),
            pltpu.SemaphoreType.DMA((8,)),
        ],
    )(partial_top, partial_bot)


QT = 256
NQ = SQ // QT


def _make_attn_body(base_tile, kv_len):
    def _attn_body(q_ref, k_ref, v_ref, o_ref):
        i = pl.program_id(1) + base_tile
        s = lax.dot_general(q_ref[...], k_ref[...], (((1,), (1,)), ((), ())),
                            preferred_element_type=jnp.float32)
        r = lax.broadcasted_iota(jnp.int32, (QT, 1), 0) + i * QT
        thr = r - lax.rem(r, BLK) + BLK
        c = lax.broadcasted_iota(jnp.int32, (QT, kv_len), 1)
        p = jnp.exp(jnp.where(c < thr, s, -1e4))
        l = jnp.sum(p, axis=1, keepdims=True)
        o = lax.dot_general(p.astype(jnp.bfloat16), v_ref[...],
                            (((1,), (0,)), ((), ())),
                            preferred_element_type=jnp.float32)
        o_ref[...] = (o / l).astype(o_ref.dtype)
    return _attn_body


def _attention_ranged(q_part, k, v, base_tile, kv_len):
    ntiles = q_part.shape[0] // QT
    return pl.pallas_call(
        _make_attn_body(base_tile, kv_len),
        out_shape=jax.ShapeDtypeStruct((ntiles * QT, H_SLICE), jnp.bfloat16),
        grid=(H_PER, ntiles),
        in_specs=[
            pl.BlockSpec((QT, DH), lambda h, i: (i, h)),
            pl.BlockSpec((kv_len, DH), lambda h, i: (0, h)),
            pl.BlockSpec((kv_len, DH), lambda h, i: (0, h)),
        ],
        out_specs=pl.BlockSpec((QT, DH), lambda h, i: (i, h)),
    )(q_part, k, v)


def kernel(x, Wq, K_ext, V_ext, Wo):
    pos = lax.axis_index("i")
    bf = jnp.bfloat16

    x2 = x[0].astype(bf)
    Wq_l = lax.dynamic_slice_in_dim(Wq, pos * H_SLICE, H_SLICE,
                                    axis=1).astype(bf)
    Wo_l = lax.dynamic_slice_in_dim(Wo, pos * H_SLICE, H_SLICE,
                                    axis=0).astype(bf)
    k = K_ext[0].reshape(SQ, H_SLICE).astype(bf)
    v = V_ext[0].reshape(SQ, H_SLICE).astype(bf)

    q = (jnp.dot(x2, Wq_l, preferred_element_type=jnp.float32)
         * SCALE).astype(bf)
    ctx_top = _attention_ranged(q[:SQ // 2], k, v, 0, SQ // 2)
    ctx_bot = _attention_ranged(q[SQ // 2:], k, v, NQ // 2, SQ)
    partial_top = jnp.dot(ctx_top, Wo_l,
                          preferred_element_type=jnp.float32)
    partial_bot = jnp.dot(ctx_bot, Wo_l,
                          preferred_element_type=jnp.float32)

    out = _ring_allreduce(partial_top, partial_bot)
    return out[None, :, :]
